# baseline (speedup 1.0000x reference)
import functools

import jax
import jax.numpy as jnp
from jax import lax
from jax.experimental import pallas as pl
from jax.experimental.pallas import tpu as pltpu
from jax.experimental.pallas import tpu_sc as plsc

NUM_BUCKETS = 100000
EMB = 16
BATCH = 16384
MAX_LEN = 100
LANE = 128

NC = 2
NS = 16
NW = NC * NS
ROWS_W = BATCH // NW
PAIRS = ROWS_W // 2
NIDX = 8
NGAT = 4
UNROLL = 8
ROWS_STAGE = NUM_BUCKETS // NS
GLEN = 104
SPL = 56


def _reduce_row(rows_v, q, j):
    accs = [rows_v[q, j, k, :] for k in range(4)]
    for l in range(4, MAX_LEN, 4):
        for k in range(4):
            accs[k] = accs[k] + rows_v[q, j, l + k, :]
    return (accs[0] + accs[1]) + (accs[2] + accs[3])


def _body(inp_hbm, tbl_hbm, out_hbm, tbl_s, idx_v, rows_v, out_v, *sems):
    isems = sems[:NIDX]
    gsems = sems[NIDX:NIDX + NGAT]
    hsems = sems[NIDX + NGAT:NIDX + 2 * NGAT]
    cid = lax.axis_index("c")
    sid = lax.axis_index("s")
    wid = sid * NC + cid
    base = wid * ROWS_W
    pbase = base // 2

    pltpu.sync_copy(tbl_hbm.at[pl.ds(sid * ROWS_STAGE, ROWS_STAGE)],
                    tbl_s.at[pl.ds(sid * ROWS_STAGE, ROWS_STAGE)])

    def fetch_idx(p, slot):
        pltpu.async_copy(inp_hbm.at[pl.ds((pbase + p) * 2, 2)],
                         idx_v.at[slot], isems[slot])

    def fire_pair(p, islot, gslot):
        pltpu.make_async_copy(inp_hbm.at[pl.ds(0, 2)], idx_v.at[islot],
                              isems[islot]).wait()
        for j in range(2):
            pltpu.async_copy(tbl_s.at[idx_v.at[islot, j, pl.ds(0, SPL)]],
                             rows_v.at[gslot, j, pl.ds(0, SPL)],
                             gsems[gslot])
            pltpu.async_copy(
                tbl_hbm.at[idx_v.at[islot, j, pl.ds(SPL, GLEN - SPL)]],
                rows_v.at[gslot, j, pl.ds(SPL, GLEN - SPL)],
                hsems[gslot])

    def drain_pair(p, gslot):
        for j in range(2):
            pltpu.make_async_copy(tbl_s.at[pl.ds(0, SPL)],
                                  rows_v.at[gslot, j, pl.ds(0, SPL)],
                                  gsems[gslot]).wait()
            pltpu.make_async_copy(tbl_hbm.at[pl.ds(0, GLEN - SPL)],
                                  rows_v.at[gslot, j, pl.ds(SPL, GLEN - SPL)],
                                  hsems[gslot]).wait()
        for j in range(2):
            acc = _reduce_row(rows_v, gslot, j)
            r = p * 2 + j
            out_v[r // 8, pl.ds((r % 8) * EMB, EMB)] = (
                acc * jnp.float32(1.0 / MAX_LEN))

    for p in range(NIDX):
        fetch_idx(p, p)
    plsc.subcore_barrier()
    for p in range(NGAT):
        fire_pair(p, p % NIDX, p % NGAT)

    def grp(g, c):
        for j in range(UNROLL):
            p = g * UNROLL + j
            drain_pair(p, j % NGAT)

            @pl.when(p + NIDX < PAIRS)
            def _():
                fetch_idx(p + NIDX, j % NIDX)

            @pl.when(p + NGAT < PAIRS)
            def _():
                fire_pair(p + NGAT, (j + NGAT) % NIDX, j % NGAT)
        return c

    lax.fori_loop(0, PAIRS // UNROLL, grp, 0)

    pltpu.sync_copy(out_v, out_hbm.at[pl.ds(wid * (ROWS_W * EMB // LANE),
                                            ROWS_W * EMB // LANE)])


_embed = functools.partial(
    pl.kernel,
    out_type=jax.ShapeDtypeStruct((BATCH * EMB // LANE, LANE), jnp.float32),
    mesh=plsc.VectorSubcoreMesh(core_axis_name="c", subcore_axis_name="s"),
    compiler_params=pltpu.CompilerParams(use_tc_tiling_on_sc=False),
    scratch_types=[
        pltpu.VMEM_SHARED((NUM_BUCKETS, EMB), jnp.float32),
        pltpu.VMEM((NIDX, 2, LANE), jnp.int32),
        pltpu.VMEM((NGAT, 2, GLEN, EMB), jnp.float32),
        pltpu.VMEM((ROWS_W * EMB // LANE, LANE), jnp.float32),
    ] + [pltpu.SemaphoreType.DMA] * (NIDX + 2 * NGAT),
)(_body)


def kernel(input, table):
    inp_p = jnp.pad(input, ((0, 0), (0, LANE - MAX_LEN)))
    out = _embed(inp_p, table)
    return out.reshape(BATCH, EMB)


# speedup vs baseline: 2.5503x; 2.5503x over previous
import functools

import jax
import jax.numpy as jnp
from jax import lax
from jax.experimental import pallas as pl
from jax.experimental.pallas import tpu as pltpu
from jax.experimental.pallas import tpu_sc as plsc

NUM_BUCKETS = 100000
EMB = 16
BATCH = 16384
MAX_LEN = 100
LANE = 128

NC = 2
NS = 16
NW = NC * NS
ROWS_W = BATCH // NW
PAIRS = ROWS_W // 2
NIDX = 8
NGAT = 4
UNROLL = 8
ROWS_STAGE = NUM_BUCKETS // NS
GLEN = 104


def _reduce_row(rows_v, q, j):
    accs = [rows_v[q, j, k, :] for k in range(4)]
    for l in range(4, MAX_LEN, 4):
        for k in range(4):
            accs[k] = accs[k] + rows_v[q, j, l + k, :]
    return (accs[0] + accs[1]) + (accs[2] + accs[3])


def _body(inp_hbm, tbl_hbm, out_hbm, tbl_s, idx_v, rows_v, out_v, *sems):
    isems = sems[:NIDX]
    gsems = sems[NIDX:NIDX + 2 * NGAT]
    cid = lax.axis_index("c")
    sid = lax.axis_index("s")
    wid = sid * NC + cid
    base = wid * ROWS_W
    pbase = base // 2

    pltpu.sync_copy(tbl_hbm.at[pl.ds(sid * ROWS_STAGE, ROWS_STAGE)],
                    tbl_s.at[pl.ds(sid * ROWS_STAGE, ROWS_STAGE)])

    def fetch_idx(p, slot):
        pltpu.async_copy(inp_hbm.at[pl.ds((pbase + p) * 2, 2)],
                         idx_v.at[slot], isems[slot])

    def fire_pair(p, islot, gslot):
        pltpu.make_async_copy(inp_hbm.at[pl.ds(0, 2)], idx_v.at[islot],
                              isems[islot]).wait()
        for j in range(2):
            pltpu.async_copy(tbl_s.at[idx_v.at[islot, j, pl.ds(0, GLEN)]],
                             rows_v.at[gslot, j], gsems[gslot * 2 + j])

    def drain_pair(p, gslot):
        for j in range(2):
            pltpu.make_async_copy(tbl_s.at[pl.ds(0, GLEN)],
                                  rows_v.at[gslot, j],
                                  gsems[gslot * 2 + j]).wait()
            acc = _reduce_row(rows_v, gslot, j)
            r = p * 2 + j
            out_v[r // 8, pl.ds((r % 8) * EMB, EMB)] = (
                acc * jnp.float32(1.0 / MAX_LEN))

    for p in range(NIDX):
        fetch_idx(p, p)
    plsc.subcore_barrier()
    for p in range(NGAT):
        fire_pair(p, p % NIDX, p % NGAT)

    def grp(g, c):
        for j in range(UNROLL):
            p = g * UNROLL + j
            drain_pair(p, j % NGAT)

            @pl.when(p + NIDX < PAIRS)
            def _():
                fetch_idx(p + NIDX, j % NIDX)

            @pl.when(p + NGAT < PAIRS)
            def _():
                fire_pair(p + NGAT, (j + NGAT) % NIDX, j % NGAT)
        return c

    lax.fori_loop(0, PAIRS // UNROLL, grp, 0)

    pltpu.sync_copy(out_v, out_hbm.at[pl.ds(wid * (ROWS_W * EMB // LANE),
                                            ROWS_W * EMB // LANE)])


_embed = functools.partial(
    pl.kernel,
    out_type=jax.ShapeDtypeStruct((BATCH * EMB // LANE, LANE), jnp.float32),
    mesh=plsc.VectorSubcoreMesh(core_axis_name="c", subcore_axis_name="s"),
    compiler_params=pltpu.CompilerParams(use_tc_tiling_on_sc=False),
    scratch_types=[
        pltpu.VMEM_SHARED((NUM_BUCKETS, EMB), jnp.float32),
        pltpu.VMEM((NIDX, 2, LANE), jnp.int32),
        pltpu.VMEM((NGAT, 2, GLEN, EMB), jnp.float32),
        pltpu.VMEM((ROWS_W * EMB // LANE, LANE), jnp.float32),
    ] + [pltpu.SemaphoreType.DMA] * (NIDX + 2 * NGAT),
)(_body)


def kernel(input, table):
    inp_p = jnp.pad(input, ((0, 0), (0, LANE - MAX_LEN)))
    out = _embed(inp_p, table)
    return out.reshape(BATCH, EMB)
